# baseline (speedup 1.0000x reference)
import functools
import math

import jax
import jax.numpy as jnp
import numpy as np
from jax import lax
from jax.experimental import pallas as pl
from jax.experimental.pallas import tpu as pltpu
from jax.experimental.pallas import tpu_sc as plsc

ALPHA = 0.4
ACCF = math.sqrt(math.log(10.0 ** 12.0))
CUTOFF = ACCF / ALPHA
KCUT = 2.0 * ALPHA * ACCF
KCUT2 = KCUT * KCUT
CONV = 1e10 * 1.602176634e-19 / (4.0 * math.pi * 8.8541878128e-12)
HALF_CONV = 0.5 * CONV
SELF_C = -ALPHA / math.sqrt(math.pi) * CONV
NEG_INV_4A2 = -1.0 / (4.0 * ALPHA * ALPHA)
NMAX = 12

AS_P = 0.3275911
AS_A1 = 0.254829592
AS_A2 = -0.284496736
AS_A3 = 1.421413741
AS_A4 = -1.453152027
AS_A5 = 1.061405429

NW = 16
L = 16
N_IMG = 64
N_PER = 256
N_ATOMS = N_IMG * N_PER
N_EDGES = N_ATOMS * 32
E_PER_W = N_EDGES // NW
GRID_PAD = 16384
G_PER_W = GRID_PAD // NW
GVECS = G_PER_W // L


def _build_grid() -> np.ndarray:
    ax = np.arange(-NMAX, NMAX + 1, dtype=np.float64)
    gx, gy, gz = np.meshgrid(ax, ax, ax, indexing="ij")
    nx, ny, nz = gx.ravel(), gy.ravel(), gz.ravel()
    arrs = [nx * nx, ny * ny, nz * nz, 2 * nx * ny, 2 * nx * nz, 2 * ny * nz]
    out = np.zeros((6, GRID_PAD), dtype=np.float32)
    for k, a in enumerate(arrs):
        out[k, : a.size] = a.astype(np.float32)
    out[0, arrs[0].size:] = 4e8
    return np.stack(
        [out[:, w * G_PER_W:(w + 1) * G_PER_W].reshape(-1) for w in range(NW)]
    )


_GRID = _build_grid()


def _build_grid_tc() -> np.ndarray:
    ax = np.arange(-NMAX, NMAX + 1, dtype=np.float64)
    gx, gy, gz = np.meshgrid(ax, ax, ax, indexing="ij")
    nx, ny, nz = gx.ravel(), gy.ravel(), gz.ravel()
    arrs = [nx * nx, ny * ny, nz * nz, 2 * nx * ny, 2 * nx * nz, 2 * ny * nz]
    out = np.zeros((6, GRID_PAD), dtype=np.float32)
    for k, a in enumerate(arrs):
        out[k, : a.size] = a.astype(np.float32)
    out[0, arrs[0].size:] = 4e8
    return out


_GRID_TC = _build_grid_tc()


def _tc_body(par_ref, grid_ref, q_ref, out_ref):
    par = par_ref[...]
    a = grid_ref[...]
    v = jax.lax.dot_general(par[:, :6], a, (((1,), (0,)), ((), ())),
                            preferred_element_type=jnp.float32)
    damp = jnp.exp(v * NEG_INV_4A2)
    val = jnp.where((v <= KCUT2) & (v > 0.0), damp / v, 0.0)
    kvec = val.sum(axis=1)
    q = q_ref[...]
    q2 = (q * q).sum(axis=1)
    out_ref[...] = (kvec * par[:, 6] + SELF_C) * q2


def _tc_kernel(par_t, grid6, qsq):
    return pl.pallas_call(
        _tc_body,
        out_shape=jax.ShapeDtypeStruct((N_IMG,), jnp.float32),
    )(par_t, grid6, qsq)


def _sc_body(d_hbm, s_hbm, o_hbm, q_hbm, out0_hbm,
             qbuf, dbuf, sbuf, obuf, acc, pvec):
    s = lax.axis_index("s")
    wid = s

    pltpu.sync_copy(q_hbm, qbuf)
    base_e = wid * E_PER_W
    pltpu.sync_copy(d_hbm.at[pl.ds(base_e, E_PER_W)], dbuf)
    pltpu.sync_copy(s_hbm.at[pl.ds(base_e, E_PER_W)], sbuf)
    pltpu.sync_copy(o_hbm.at[pl.ds(base_e, E_PER_W)], obuf)

    lane = lax.iota(jnp.int32, L)
    lane64 = lane * 64
    zeros16 = jnp.zeros((L,), jnp.float32)

    def z_body(i, _):
        acc[pl.ds(i * L, L)] = zeros16
        return _
    lax.fori_loop(0, N_IMG, z_body, None)

    def e_body(j, _):
        b = j * L
        d = dbuf[pl.ds(b, L)]
        si = sbuf[pl.ds(b, L)]
        oi = obuf[pl.ds(b, L)]
        qs = plsc.load_gather(qbuf, [si])
        qo = plsc.load_gather(qbuf, [oi])
        x = ALPHA * d
        t = 1.0 / (1.0 + AS_P * x)
        poly = t * (AS_A1 + t * (AS_A2 + t * (AS_A3 + t * (AS_A4 + t * AS_A5))))
        w = poly * jnp.exp(-(x * x)) / d
        contrib = qs * qo * w
        idx = lane64 + lax.shift_right_logical(si, 8)
        plsc.addupdate_scatter(acc, [idx], contrib, mask=d < CUTOFF)
        return _
    lax.fori_loop(0, E_PER_W // L, e_body, None, unroll=4)

    def f_body(blk, _):
        def r_body(r, v):
            return v + acc[pl.ds(r * 64 + blk * L, L)]
        v = lax.fori_loop(0, L, r_body, zeros16)
        pvec[pl.ds(blk * L, L)] = v * HALF_CONV
        return _
    lax.fori_loop(0, 4, f_body, None)

    pltpu.sync_copy(pvec, out0_hbm.at[s])


@functools.lru_cache(maxsize=1)
def _get_sc_kernel():
    return functools.partial(
        pl.kernel,
        out_type=jax.ShapeDtypeStruct((16, N_IMG), jnp.float32),
        mesh=plsc.VectorSubcoreMesh(core_axis_name="c", subcore_axis_name="s",
                                    num_cores=1, num_subcores=16),
        compiler_params=pltpu.CompilerParams(needs_layout_passes=False,
                                             skip_device_barrier=True),
        scratch_types=[
            pltpu.VMEM((N_ATOMS,), jnp.float32),
            pltpu.VMEM((E_PER_W,), jnp.float32),
            pltpu.VMEM((E_PER_W,), jnp.int32),
            pltpu.VMEM((E_PER_W,), jnp.int32),
            pltpu.VMEM((16 * N_IMG,), jnp.float32),
            pltpu.VMEM((N_IMG,), jnp.float32),
        ],
    )(_sc_body)


def kernel(edge_dist, edge_idx, atomic_charge, cell, n_atoms, positions, image_idx):
    cells = cell.reshape(-1, 3, 3)
    seg = edge_idx[:, 0].astype(jnp.int32)
    oth = edge_idx[:, 1].astype(jnp.int32)

    recip = 2.0 * math.pi * jnp.linalg.inv(cells).transpose(0, 2, 1)
    gram = jnp.einsum("nki,nkj->nij", recip, recip)
    vols = jnp.sum(cells[:, 0] * jnp.cross(cells[:, 1], cells[:, 2]), axis=1)
    prefc = CONV / (2.0 * math.pi * vols)
    par_t = jnp.stack(
        [gram[:, 0, 0], gram[:, 1, 1], gram[:, 2, 2],
         2.0 * gram[:, 0, 1], 2.0 * gram[:, 0, 2], 2.0 * gram[:, 1, 2],
         prefc, jnp.zeros_like(prefc)], axis=1)

    rvec = _tc_kernel(par_t, jnp.asarray(_GRID_TC),
                      atomic_charge.reshape(N_IMG, N_PER))
    p0 = _get_sc_kernel()(edge_dist, seg, oth, atomic_charge)
    return p0.sum(axis=0) + rvec


# speedup vs baseline: 1.1201x; 1.1201x over previous
import functools
import math

import jax
import jax.numpy as jnp
import numpy as np
from jax import lax
from jax.experimental import pallas as pl
from jax.experimental.pallas import tpu as pltpu
from jax.experimental.pallas import tpu_sc as plsc

ALPHA = 0.4
ACCF = math.sqrt(math.log(10.0 ** 12.0))
CUTOFF = ACCF / ALPHA
KCUT = 2.0 * ALPHA * ACCF
KCUT2 = KCUT * KCUT
CONV = 1e10 * 1.602176634e-19 / (4.0 * math.pi * 8.8541878128e-12)
HALF_CONV = 0.5 * CONV
SELF_C = -ALPHA / math.sqrt(math.pi) * CONV
NEG_INV_4A2 = -1.0 / (4.0 * ALPHA * ALPHA)
NMAX = 12

AS_P = 0.3275911
AS_A1 = 0.254829592
AS_A2 = -0.284496736
AS_A3 = 1.421413741
AS_A4 = -1.453152027
AS_A5 = 1.061405429

NW = 32
L = 16
N_IMG = 64
N_PER = 256
N_ATOMS = N_IMG * N_PER
N_EDGES = N_ATOMS * 32
E_PER_W = N_EDGES // NW
GRID_PAD = 16384
G_PER_W = GRID_PAD // NW
GVECS = G_PER_W // L


def _build_grid() -> np.ndarray:
    ax = np.arange(-NMAX, NMAX + 1, dtype=np.float64)
    gx, gy, gz = np.meshgrid(ax, ax, ax, indexing="ij")
    nx, ny, nz = gx.ravel(), gy.ravel(), gz.ravel()
    arrs = [nx * nx, ny * ny, nz * nz, 2 * nx * ny, 2 * nx * nz, 2 * ny * nz]
    out = np.zeros((6, GRID_PAD), dtype=np.float32)
    for k, a in enumerate(arrs):
        out[k, : a.size] = a.astype(np.float32)
    out[0, arrs[0].size:] = 4e8
    return np.stack(
        [out[:, w * G_PER_W:(w + 1) * G_PER_W].reshape(-1) for w in range(NW)]
    )


_GRID = _build_grid()


def _build_grid_tc() -> np.ndarray:
    ax = np.arange(-NMAX, NMAX + 1, dtype=np.float64)
    gx, gy, gz = np.meshgrid(ax, ax, ax, indexing="ij")
    nx, ny, nz = gx.ravel(), gy.ravel(), gz.ravel()
    arrs = [nx * nx, ny * ny, nz * nz, 2 * nx * ny, 2 * nx * nz, 2 * ny * nz]
    out = np.zeros((6, GRID_PAD), dtype=np.float32)
    for k, a in enumerate(arrs):
        out[k, : a.size] = a.astype(np.float32)
    out[0, arrs[0].size:] = 4e8
    return out


_GRID_TC = _build_grid_tc()


def _tc_body(par_ref, grid_ref, q_ref, out_ref):
    par = par_ref[...]
    a = grid_ref[...]
    v = jax.lax.dot_general(par[:, :6], a, (((1,), (0,)), ((), ())),
                            preferred_element_type=jnp.float32)
    damp = jnp.exp(v * NEG_INV_4A2)
    val = jnp.where((v <= KCUT2) & (v > 0.0), damp / v, 0.0)
    kvec = val.sum(axis=1)
    q = q_ref[...]
    q2 = (q * q).sum(axis=1)
    out_ref[...] = (kvec * par[:, 6] + SELF_C) * q2


def _tcw_body(d_ref, w_ref):
    d = d_ref[...]
    x = ALPHA * d
    t = 1.0 / (1.0 + AS_P * x)
    poly = t * (AS_A1 + t * (AS_A2 + t * (AS_A3 + t * (AS_A4 + t * AS_A5))))
    w = poly * jnp.exp(-(x * x)) / d
    w_ref[...] = jnp.where(d < CUTOFF, w, 0.0)


def _tcw_kernel(d2):
    return pl.pallas_call(
        _tcw_body,
        out_shape=jax.ShapeDtypeStruct(d2.shape, jnp.float32),
    )(d2)


def _tc_kernel(par_t, grid6, qsq):
    return pl.pallas_call(
        _tc_body,
        out_shape=jax.ShapeDtypeStruct((N_IMG,), jnp.float32),
    )(par_t, grid6, qsq)


def _sc_body(d_hbm, s_hbm, o_hbm, q_hbm, out0_hbm, out1_hbm,
             qbuf, dbuf, sbuf, obuf, acc, pvec):
    c = lax.axis_index("c")
    s = lax.axis_index("s")
    wid = c * 16 + s

    pltpu.sync_copy(q_hbm, qbuf)
    base_e = wid * E_PER_W
    pltpu.sync_copy(d_hbm.at[pl.ds(base_e, E_PER_W)], dbuf)
    pltpu.sync_copy(s_hbm.at[pl.ds(base_e, E_PER_W)], sbuf)
    pltpu.sync_copy(o_hbm.at[pl.ds(base_e, E_PER_W)], obuf)

    lane = lax.iota(jnp.int32, L)
    lane64 = lane * 64
    zeros16 = jnp.zeros((L,), jnp.float32)

    def z_body(i, _):
        acc[pl.ds(i * L, L)] = zeros16
        return _
    lax.fori_loop(0, N_IMG, z_body, None)

    def e_body(j, _):
        b = j * L
        wv = dbuf[pl.ds(b, L)]
        si = sbuf[pl.ds(b, L)]
        oi = obuf[pl.ds(b, L)]
        qs = plsc.load_gather(qbuf, [si])
        qo = plsc.load_gather(qbuf, [oi])
        contrib = qs * qo * wv
        idx = lane64 + lax.shift_right_logical(si, 8)
        plsc.addupdate_scatter(acc, [idx], contrib)
        return _
    lax.fori_loop(0, E_PER_W // L, e_body, None, unroll=4)

    def f_body(blk, _):
        def r_body(r, v):
            return v + acc[pl.ds(r * 64 + blk * L, L)]
        v = lax.fori_loop(0, L, r_body, zeros16)
        pvec[pl.ds(blk * L, L)] = v * HALF_CONV
        return _
    lax.fori_loop(0, 4, f_body, None)

    @pl.when(c == 0)
    def _():
        pltpu.sync_copy(pvec, out0_hbm.at[s])

    @pl.when(c == 1)
    def _():
        pltpu.sync_copy(pvec, out1_hbm.at[s])


@functools.lru_cache(maxsize=1)
def _get_sc_kernel():
    return functools.partial(
        pl.kernel,
        out_type=(jax.ShapeDtypeStruct((16, N_IMG), jnp.float32),
                  jax.ShapeDtypeStruct((16, N_IMG), jnp.float32)),
        mesh=plsc.VectorSubcoreMesh(core_axis_name="c", subcore_axis_name="s",
                                    num_cores=2, num_subcores=16),
        compiler_params=pltpu.CompilerParams(needs_layout_passes=False,
                                             skip_device_barrier=True),
        scratch_types=[
            pltpu.VMEM((N_ATOMS,), jnp.float32),
            pltpu.VMEM((E_PER_W,), jnp.float32),
            pltpu.VMEM((E_PER_W,), jnp.int32),
            pltpu.VMEM((E_PER_W,), jnp.int32),
            pltpu.VMEM((16 * N_IMG,), jnp.float32),
            pltpu.VMEM((N_IMG,), jnp.float32),
        ],
    )(_sc_body)


def kernel(edge_dist, edge_idx, atomic_charge, cell, n_atoms, positions, image_idx):
    cells = cell.reshape(-1, 3, 3)
    seg = edge_idx[:, 0].astype(jnp.int32)
    oth = edge_idx[:, 1].astype(jnp.int32)

    recip = 2.0 * math.pi * jnp.linalg.inv(cells).transpose(0, 2, 1)
    gram = jnp.einsum("nki,nkj->nij", recip, recip)
    vols = jnp.sum(cells[:, 0] * jnp.cross(cells[:, 1], cells[:, 2]), axis=1)
    prefc = CONV / (2.0 * math.pi * vols)
    par_t = jnp.stack(
        [gram[:, 0, 0], gram[:, 1, 1], gram[:, 2, 2],
         2.0 * gram[:, 0, 1], 2.0 * gram[:, 0, 2], 2.0 * gram[:, 1, 2],
         prefc, jnp.zeros_like(prefc)], axis=1)

    rvec = _tc_kernel(par_t, jnp.asarray(_GRID_TC),
                      atomic_charge.reshape(N_IMG, N_PER))
    wflat = _tcw_kernel(edge_dist.reshape(4096, 128)).reshape(-1)
    p0, p1 = _get_sc_kernel()(wflat, seg, oth, atomic_charge)
    return p0.sum(axis=0) + p1.sum(axis=0) + rvec


# speedup vs baseline: 1.1275x; 1.0066x over previous
import functools
import math

import jax
import jax.numpy as jnp
import numpy as np
from jax import lax
from jax.experimental import pallas as pl
from jax.experimental.pallas import tpu as pltpu
from jax.experimental.pallas import tpu_sc as plsc

ALPHA = 0.4
ACCF = math.sqrt(math.log(10.0 ** 12.0))
CUTOFF = ACCF / ALPHA
KCUT = 2.0 * ALPHA * ACCF
KCUT2 = KCUT * KCUT
CONV = 1e10 * 1.602176634e-19 / (4.0 * math.pi * 8.8541878128e-12)
HALF_CONV = 0.5 * CONV
SELF_C = -ALPHA / math.sqrt(math.pi) * CONV
NEG_INV_4A2 = -1.0 / (4.0 * ALPHA * ALPHA)
NMAX = 12

AS_P = 0.3275911
AS_A1 = 0.254829592
AS_A2 = -0.284496736
AS_A3 = 1.421413741
AS_A4 = -1.453152027
AS_A5 = 1.061405429

NW = 32
L = 16
N_IMG = 64
N_PER = 256
N_ATOMS = N_IMG * N_PER
N_EDGES = N_ATOMS * 32
E_PER_W = N_EDGES // NW
GRID_PAD = 16384
G_PER_W = GRID_PAD // NW
GVECS = G_PER_W // L


def _build_grid() -> np.ndarray:
    ax = np.arange(-NMAX, NMAX + 1, dtype=np.float64)
    gx, gy, gz = np.meshgrid(ax, ax, ax, indexing="ij")
    nx, ny, nz = gx.ravel(), gy.ravel(), gz.ravel()
    arrs = [nx * nx, ny * ny, nz * nz, 2 * nx * ny, 2 * nx * nz, 2 * ny * nz]
    out = np.zeros((6, GRID_PAD), dtype=np.float32)
    for k, a in enumerate(arrs):
        out[k, : a.size] = a.astype(np.float32)
    out[0, arrs[0].size:] = 4e8
    return np.stack(
        [out[:, w * G_PER_W:(w + 1) * G_PER_W].reshape(-1) for w in range(NW)]
    )


_GRID = _build_grid()


def _build_grid_tc() -> np.ndarray:
    ax = np.arange(-NMAX, NMAX + 1, dtype=np.float64)
    gx, gy, gz = np.meshgrid(ax, ax, ax, indexing="ij")
    nx, ny, nz = gx.ravel(), gy.ravel(), gz.ravel()
    arrs = [nx * nx, ny * ny, nz * nz, 2 * nx * ny, 2 * nx * nz, 2 * ny * nz]
    out = np.zeros((6, GRID_PAD), dtype=np.float32)
    for k, a in enumerate(arrs):
        out[k, : a.size] = a.astype(np.float32)
    out[0, arrs[0].size:] = 4e8
    return out


_GRID_TC = _build_grid_tc()


def _tc_body(par_ref, grid_ref, q_ref, out_ref):
    par = par_ref[...]
    a = grid_ref[...]
    v = jax.lax.dot_general(par[:, :6], a, (((1,), (0,)), ((), ())),
                            preferred_element_type=jnp.float32)
    damp = jnp.exp(v * NEG_INV_4A2)
    val = jnp.where((v <= KCUT2) & (v > 0.0), damp / v, 0.0)
    kvec = val.sum(axis=1)
    q = q_ref[...]
    q2 = (q * q).sum(axis=1)
    out_ref[...] = (kvec * par[:, 6] + SELF_C) * q2


def _tc_kernel(par_t, grid6, qsq):
    return pl.pallas_call(
        _tc_body,
        out_shape=jax.ShapeDtypeStruct((N_IMG,), jnp.float32),
    )(par_t, grid6, qsq)


def _sc_body(d_hbm, s_hbm, o_hbm, q_hbm, out0_hbm, out1_hbm,
             qbuf, dbuf, sbuf, obuf, acc, pvec):
    c = lax.axis_index("c")
    s = lax.axis_index("s")
    wid = c * 16 + s

    pltpu.sync_copy(q_hbm, qbuf)
    base_e = wid * E_PER_W
    pltpu.sync_copy(d_hbm.at[pl.ds(base_e, E_PER_W)], dbuf)
    pltpu.sync_copy(s_hbm.at[pl.ds(base_e, E_PER_W)], sbuf)
    pltpu.sync_copy(o_hbm.at[pl.ds(base_e, E_PER_W)], obuf)

    lane = lax.iota(jnp.int32, L)
    lane64 = lane * 64
    zeros16 = jnp.zeros((L,), jnp.float32)

    def z_body(i, _):
        acc[pl.ds(i * L, L)] = zeros16
        return _
    lax.fori_loop(0, N_IMG, z_body, None)

    def e_body(j, _):
        b = j * L
        d = dbuf[pl.ds(b, L)]
        si = sbuf[pl.ds(b, L)]
        oi = obuf[pl.ds(b, L)]
        qs = plsc.load_gather(qbuf, [si])
        qo = plsc.load_gather(qbuf, [oi])
        x = ALPHA * d
        t = 1.0 / (1.0 + AS_P * x)
        poly = t * (AS_A1 + t * (AS_A2 + t * (AS_A3 + t * (AS_A4 + t * AS_A5))))
        w = poly * jnp.exp(-(x * x)) / d
        contrib = qs * qo * w
        idx = lane64 + lax.shift_right_logical(si, 8)
        plsc.addupdate_scatter(acc, [idx], contrib, mask=d < CUTOFF)
        return _
    lax.fori_loop(0, E_PER_W // L, e_body, None, unroll=4)

    def f_body(blk, _):
        def r_body(r, v):
            return v + acc[pl.ds(r * 64 + blk * L, L)]
        v = lax.fori_loop(0, L, r_body, zeros16)
        pvec[pl.ds(blk * L, L)] = v * HALF_CONV
        return _
    lax.fori_loop(0, 4, f_body, None)

    @pl.when(c == 0)
    def _():
        pltpu.sync_copy(pvec, out0_hbm.at[s])

    @pl.when(c == 1)
    def _():
        pltpu.sync_copy(pvec, out1_hbm.at[s])


@functools.lru_cache(maxsize=1)
def _get_sc_kernel():
    return functools.partial(
        pl.kernel,
        out_type=(jax.ShapeDtypeStruct((16, N_IMG), jnp.float32),
                  jax.ShapeDtypeStruct((16, N_IMG), jnp.float32)),
        mesh=plsc.VectorSubcoreMesh(core_axis_name="c", subcore_axis_name="s",
                                    num_cores=2, num_subcores=16),
        compiler_params=pltpu.CompilerParams(needs_layout_passes=False,
                                             skip_device_barrier=True),
        scratch_types=[
            pltpu.VMEM((N_ATOMS,), jnp.float32),
            pltpu.VMEM((E_PER_W,), jnp.float32),
            pltpu.VMEM((E_PER_W,), jnp.int32),
            pltpu.VMEM((E_PER_W,), jnp.int32),
            pltpu.VMEM((16 * N_IMG,), jnp.float32),
            pltpu.VMEM((N_IMG,), jnp.float32),
        ],
    )(_sc_body)


def kernel(edge_dist, edge_idx, atomic_charge, cell, n_atoms, positions, image_idx):
    cells = cell.reshape(-1, 3, 3)
    seg = edge_idx[:, 0].astype(jnp.int32)
    oth = edge_idx[:, 1].astype(jnp.int32)

    recip = 2.0 * math.pi * jnp.linalg.inv(cells).transpose(0, 2, 1)
    gram = jnp.einsum("nki,nkj->nij", recip, recip)
    vols = jnp.sum(cells[:, 0] * jnp.cross(cells[:, 1], cells[:, 2]), axis=1)
    prefc = CONV / (2.0 * math.pi * vols)
    par_t = jnp.stack(
        [gram[:, 0, 0], gram[:, 1, 1], gram[:, 2, 2],
         2.0 * gram[:, 0, 1], 2.0 * gram[:, 0, 2], 2.0 * gram[:, 1, 2],
         prefc, jnp.zeros_like(prefc)], axis=1)

    rvec = _tc_kernel(par_t, jnp.asarray(_GRID_TC),
                      atomic_charge.reshape(N_IMG, N_PER))
    p0, p1 = _get_sc_kernel()(edge_dist, seg, oth, atomic_charge)
    return p0.sum(axis=0) + p1.sum(axis=0) + rvec
